# baseline (speedup 1.0000x reference)
import functools

import jax
import jax.numpy as jnp
from jax import lax
from jax.experimental import pallas as pl
from jax.experimental.pallas import tpu as pltpu
from jax.experimental.pallas import tpu_sc as plsc

N_CLASSES = 1000
N_CTX = 8
NAME_LEN = 77
OUT_LEN = N_CTX + NAME_LEN
CTX_DIM = 512

B = 5
RD = 8
NCHUNKS = N_CLASSES // B
PER_CORE = NCHUNKS // 2


def kernel(ctx, name_embeds):
    mesh = plsc.ScalarSubcoreMesh(axis_name="c", num_cores=2)

    @functools.partial(
        pl.kernel,
        mesh=mesh,
        out_type=jax.ShapeDtypeStruct((N_CLASSES, OUT_LEN, CTX_DIM), jnp.float32),
        scratch_types=[
            pltpu.VMEM_SHARED((RD, B, OUT_LEN, CTX_DIM), jnp.float32),
            pltpu.SemaphoreType.DMA((RD, 2)),
            pltpu.SemaphoreType.DMA((RD,)),
        ],
    )
    def _sc_concat(ctx_hbm, name_hbm, out_hbm, buf, gsems, ssems):
        cid = lax.axis_index("c")
        base = cid * PER_CORE

        def gathers(chunk, slot):
            c0 = chunk * B
            g1 = pltpu.make_async_copy(
                ctx_hbm.at[pl.ds(c0, B)],
                buf.at[slot, :, pl.ds(0, N_CTX)],
                gsems.at[slot, 0],
            )
            g2 = pltpu.make_async_copy(
                name_hbm.at[pl.ds(c0, B)],
                buf.at[slot, :, pl.ds(N_CTX, NAME_LEN)],
                gsems.at[slot, 1],
            )
            return g1, g2

        def scatter(chunk, slot):
            return pltpu.make_async_copy(
                buf.at[slot], out_hbm.at[pl.ds(chunk * B, B)], ssems.at[slot]
            )

        def body(k, _):
            slot = k % RD

            @pl.when(k >= RD)
            def _():
                scatter(base + k - RD, slot).wait()

            @pl.when(k < PER_CORE)
            def _():
                g1, g2 = gathers(base + k, slot)
                g1.start()
                g2.start()

            @pl.when((k >= 1) & (k <= PER_CORE))
            def _():
                prev = (k - 1) % RD
                g1, g2 = gathers(base + k - 1, prev)
                g1.wait()
                g2.wait()
                scatter(base + k - 1, prev).start()

            return 0

        lax.fori_loop(0, PER_CORE + 2, body, 0)
        scatter(base + PER_CORE - 2, (PER_CORE - 2) % RD).wait()
        scatter(base + PER_CORE - 1, (PER_CORE - 1) % RD).wait()

    return _sc_concat(ctx, name_embeds)


# speedup vs baseline: 1.0309x; 1.0309x over previous
import functools

import jax
import jax.numpy as jnp
from jax import lax
from jax.experimental import pallas as pl
from jax.experimental.pallas import tpu as pltpu
from jax.experimental.pallas import tpu_sc as plsc

N_CLASSES = 1000
N_CTX = 8
NAME_LEN = 77
OUT_LEN = N_CTX + NAME_LEN
CTX_DIM = 512

B = 10
RD = 4
NCHUNKS = N_CLASSES // B
PER_CORE = NCHUNKS // 2


def kernel(ctx, name_embeds):
    mesh = plsc.ScalarSubcoreMesh(axis_name="c", num_cores=2)

    @functools.partial(
        pl.kernel,
        mesh=mesh,
        out_type=jax.ShapeDtypeStruct((N_CLASSES, OUT_LEN, CTX_DIM), jnp.float32),
        scratch_types=[
            pltpu.VMEM_SHARED((RD, B, OUT_LEN, CTX_DIM), jnp.float32),
            pltpu.SemaphoreType.DMA((RD, 2)),
            pltpu.SemaphoreType.DMA((RD,)),
        ],
    )
    def _sc_concat(ctx_hbm, name_hbm, out_hbm, buf, gsems, ssems):
        cid = lax.axis_index("c")
        base = cid * PER_CORE

        def gathers(chunk, slot):
            c0 = chunk * B
            g1 = pltpu.make_async_copy(
                ctx_hbm.at[pl.ds(c0, B)],
                buf.at[slot, :, pl.ds(0, N_CTX)],
                gsems.at[slot, 0],
            )
            g2 = pltpu.make_async_copy(
                name_hbm.at[pl.ds(c0, B)],
                buf.at[slot, :, pl.ds(N_CTX, NAME_LEN)],
                gsems.at[slot, 1],
            )
            return g1, g2

        def scatter(chunk, slot):
            return pltpu.make_async_copy(
                buf.at[slot], out_hbm.at[pl.ds(chunk * B, B)], ssems.at[slot]
            )

        def body(k, _):
            slot = k % RD

            @pl.when(k >= RD)
            def _():
                scatter(base + k - RD, slot).wait()

            @pl.when(k < PER_CORE)
            def _():
                g1, g2 = gathers(base + k, slot)
                g1.start()
                g2.start()

            @pl.when((k >= 1) & (k <= PER_CORE))
            def _():
                prev = (k - 1) % RD
                g1, g2 = gathers(base + k - 1, prev)
                g1.wait()
                g2.wait()
                scatter(base + k - 1, prev).start()

            return 0

        lax.fori_loop(0, PER_CORE + 2, body, 0)
        scatter(base + PER_CORE - 2, (PER_CORE - 2) % RD).wait()
        scatter(base + PER_CORE - 1, (PER_CORE - 1) % RD).wait()

    return _sc_concat(ctx, name_embeds)
